# baseline (speedup 1.0000x reference)
import numpy as np
import jax
import jax.numpy as jnp
from jax.experimental import pallas as pl
from jax.experimental.pallas import tpu as pltpu
from jax._src.pallas.core import Element

PS = 16
STEP = 14
HEADS = 4
DIM = 96
QK_DIM = 64
LN_EPS = 1e-5
H = 224
NPATCH = 16
NTOK = PS * PS
_dq = QK_DIM // HEADS
_dv = DIM // HEADS

OFFS = [min(STEP * i, H - PS) for i in range(NPATCH)]

BANDS = []
for _i in range(STEP, H + STEP - PS, STEP):
    _top = _i
    _down = _i + PS - STEP
    if _top + PS > H:
        _top = H - PS
    BANDS.append((_top, _down))


def _band_weights(idx):
    band = (idx < 0)
    for a, b in BANDS:
        band = band | ((idx >= a) & (idx < b))
    return jnp.where(band, jnp.float32(0.5), jnp.float32(1.0))


def _top_of(i):
    return jnp.minimum(STEP * i, H - PS)


def _qkv_kernel(x_ref, ln_g_ref, ln_b_ref, wq_ref, wk_ref, wv_ref,
                q_ref, k_ref, v_ref):
    t = x_ref[...].T
    mu = jnp.mean(t, axis=-1, keepdims=True)
    d = t - mu
    var = jnp.mean(d * d, axis=-1, keepdims=True)
    xn = (d * jax.lax.rsqrt(var + LN_EPS) * ln_g_ref[...]
          + ln_b_ref[...]).astype(jnp.bfloat16)
    q_ref[...] = jnp.dot(xn, wq_ref[...],
                         preferred_element_type=jnp.float32
                         ).astype(jnp.bfloat16)
    k_ref[...] = jnp.dot(xn, wk_ref[...],
                         preferred_element_type=jnp.float32
                         ).astype(jnp.bfloat16)
    v_ref[...] = jnp.dot(xn, wv_ref[...],
                         preferred_element_type=jnp.float32
                         ).astype(jnp.bfloat16)


def _attn_kernel(q_ref, k_ref, v_ref, o_ref):
    qrow = jax.lax.broadcasted_iota(jnp.int32, (HEADS * NTOK, 1), 0) // NTOK
    qcol = jax.lax.broadcasted_iota(jnp.int32, (1, QK_DIM), 1) // _dq
    stackmask = (qrow == qcol).astype(jnp.bfloat16)
    vi = jax.lax.broadcasted_iota(jnp.int32, (1, DIM), 1) // _dv

    o_list = []
    for iw in range(NPATCH):
        left = OFFS[iw]
        q_p = q_ref[:, left:left + PS, :].reshape(NTOK, QK_DIM)
        k_p = k_ref[:, left:left + PS, :].reshape(NTOK, QK_DIM)
        v_p = v_ref[:, left:left + PS, :].reshape(NTOK, DIM)

        q_stack = jnp.concatenate([q_p] * HEADS, axis=0) * stackmask
        s = jax.lax.dot_general(
            q_stack, k_p, (((1,), (1,)), ((), ())),
            precision=jax.lax.Precision.DEFAULT,
            preferred_element_type=jnp.float32)
        s = s - jnp.max(s, axis=-1, keepdims=True)
        e = jnp.exp(s)
        a = (e / jnp.sum(e, axis=-1, keepdims=True)).astype(jnp.bfloat16)
        ov = jnp.dot(a, v_p, precision=jax.lax.Precision.DEFAULT,
                     preferred_element_type=jnp.float32)
        o = (vi == 0).astype(jnp.float32) * ov[0:NTOK]
        for h in range(1, HEADS):
            o = o + (vi == h).astype(jnp.float32) * ov[h * NTOK:(h + 1) * NTOK]
        o_list.append(o)

    o_ref[...] = jnp.concatenate(o_list, axis=0).reshape(
        1, NPATCH * NTOK, DIM).astype(jnp.bfloat16)


def _proj_kernel(o_ref, wp_ref, out_ref):
    i = pl.program_id(0)
    top = _top_of(i)

    o_row = o_ref[...].reshape(NPATCH * NTOK, DIM)
    res_row = jnp.dot(o_row, wp_ref[...],
                      preferred_element_type=jnp.float32)

    wr = _band_weights(
        top + jax.lax.broadcasted_iota(jnp.int32, (PS, 1, 1), 0))
    colw = _band_weights(
        jax.lax.broadcasted_iota(jnp.int32, (1, H, 1), 1))

    patches = []
    for iw in range(NPATCH):
        left = OFFS[iw]
        p = res_row[iw * NTOK:(iw + 1) * NTOK].reshape(PS, PS, DIM)
        patches.append(p * wr * colw[:, left:left + PS, :])

    pieces = [patches[0][:, 0:OFFS[1], :]]
    for iw in range(1, NPATCH):
        ov_w = OFFS[iw - 1] + PS - OFFS[iw]
        pieces.append(patches[iw - 1][:, PS - ov_w:PS, :]
                      + patches[iw][:, 0:ov_w, :])
        hi = (OFFS[iw + 1] - OFFS[iw]) if iw + 1 < NPATCH else PS
        pieces.append(patches[iw][:, ov_w:hi, :])
    slab = jnp.concatenate(pieces, axis=1)

    @pl.when(i == 0)
    def _zero():
        out_ref[...] = jnp.zeros_like(out_ref)

    out_ref[pl.ds(top, PS)] += slab


def _back_kernel(y_ref, x_ref, out_ref):
    out_ref[...] = y_ref[...].T + x_ref[...]


def kernel(x, ln_g, ln_b, Wq, Wk, Wv, Wp):
    x2 = x[0].reshape(DIM, H * H)
    wq_s = Wq * (1.0 / np.sqrt(_dq))

    nrow = H * H // 14
    q, k, v = pl.pallas_call(
        _qkv_kernel,
        grid=(14,),
        in_specs=[
            pl.BlockSpec((DIM, nrow), lambda i: (0, i)),
            pl.BlockSpec((1, DIM), lambda i: (0, 0)),
            pl.BlockSpec((1, DIM), lambda i: (0, 0)),
            pl.BlockSpec((DIM, QK_DIM), lambda i: (0, 0)),
            pl.BlockSpec((DIM, QK_DIM), lambda i: (0, 0)),
            pl.BlockSpec((DIM, DIM), lambda i: (0, 0)),
        ],
        out_specs=[
            pl.BlockSpec((nrow, QK_DIM), lambda i: (i, 0)),
            pl.BlockSpec((nrow, QK_DIM), lambda i: (i, 0)),
            pl.BlockSpec((nrow, DIM), lambda i: (i, 0)),
        ],
        out_shape=[
            jax.ShapeDtypeStruct((H * H, QK_DIM), jnp.bfloat16),
            jax.ShapeDtypeStruct((H * H, QK_DIM), jnp.bfloat16),
            jax.ShapeDtypeStruct((H * H, DIM), jnp.bfloat16),
        ],
    )(x2, ln_g.reshape(1, DIM), ln_b.reshape(1, DIM),
      wq_s.astype(jnp.bfloat16), Wk.astype(jnp.bfloat16),
      Wv.astype(jnp.bfloat16))

    q4 = q.reshape(H, H, QK_DIM)
    k4 = k.reshape(H, H, QK_DIM)
    v4 = v.reshape(H, H, DIM)

    o_all = pl.pallas_call(
        _attn_kernel,
        grid=(NPATCH,),
        in_specs=[
            pl.BlockSpec((Element(PS), Element(H), Element(QK_DIM)),
                         lambda i: (_top_of(i), 0, 0)),
            pl.BlockSpec((Element(PS), Element(H), Element(QK_DIM)),
                         lambda i: (_top_of(i), 0, 0)),
            pl.BlockSpec((Element(PS), Element(H), Element(DIM)),
                         lambda i: (_top_of(i), 0, 0)),
        ],
        out_specs=pl.BlockSpec((1, NPATCH * NTOK, DIM), lambda i: (i, 0, 0)),
        out_shape=jax.ShapeDtypeStruct((NPATCH, NPATCH * NTOK, DIM),
                                       jnp.bfloat16),
    )(q4, k4, v4)

    out_hwc = pl.pallas_call(
        _proj_kernel,
        grid=(NPATCH,),
        in_specs=[
            pl.BlockSpec((1, NPATCH * NTOK, DIM), lambda i: (i, 0, 0)),
            pl.BlockSpec((DIM, DIM), lambda i: (0, 0)),
        ],
        out_specs=pl.BlockSpec((H, H, DIM), lambda i: (0, 0, 0)),
        out_shape=jax.ShapeDtypeStruct((H, H, DIM), jnp.float32),
    )(o_all, Wp.astype(jnp.bfloat16))

    out_chw = pl.pallas_call(
        _back_kernel,
        grid=(14,),
        in_specs=[
            pl.BlockSpec((nrow, DIM), lambda i: (i, 0)),
            pl.BlockSpec((DIM, nrow), lambda i: (0, i)),
        ],
        out_specs=pl.BlockSpec((DIM, nrow), lambda i: (0, i)),
        out_shape=jax.ShapeDtypeStruct((DIM, H * H), jnp.float32),
    )(out_hwc.reshape(H * H, DIM), x2)

    return out_chw.reshape(1, DIM, H, H)


# speedup vs baseline: 1.2405x; 1.2405x over previous
import numpy as np
import jax
import jax.numpy as jnp
from jax.experimental import pallas as pl
from jax.experimental.pallas import tpu as pltpu
from jax._src.pallas.core import Element

PS = 16
STEP = 14
HEADS = 4
DIM = 96
QK_DIM = 64
LN_EPS = 1e-5
H = 224
NPATCH = 16
NTOK = PS * PS
_dq = QK_DIM // HEADS
_dv = DIM // HEADS

OFFS = [min(STEP * i, H - PS) for i in range(NPATCH)]

BANDS = []
for _i in range(STEP, H + STEP - PS, STEP):
    _top = _i
    _down = _i + PS - STEP
    if _top + PS > H:
        _top = H - PS
    BANDS.append((_top, _down))


def _band_weights(idx):
    band = (idx < 0)
    for a, b in BANDS:
        band = band | ((idx >= a) & (idx < b))
    return jnp.where(band, jnp.float32(0.5), jnp.float32(1.0))


def _top_of(i):
    return jnp.minimum(STEP * i, H - PS)


def _qkv_kernel(x_ref, ln_g_ref, ln_b_ref, wq_ref, wk_ref, wv_ref,
                q_ref, k_ref, v_ref):
    t = x_ref[...].T
    mu = jnp.mean(t, axis=-1, keepdims=True)
    d = t - mu
    var = jnp.mean(d * d, axis=-1, keepdims=True)
    xn = d * jax.lax.rsqrt(var + LN_EPS) * ln_g_ref[...] + ln_b_ref[...]
    q_ref[...] = jnp.dot(xn, wq_ref[...], preferred_element_type=jnp.float32)
    k_ref[...] = jnp.dot(xn, wk_ref[...], preferred_element_type=jnp.float32)
    v_ref[...] = jnp.dot(xn, wv_ref[...], preferred_element_type=jnp.float32)


def _attn_kernel(q_ref, k_ref, v_ref, o_ref):
    qrow = jax.lax.broadcasted_iota(jnp.int32, (HEADS * NTOK, 1), 0) // NTOK
    qcol = jax.lax.broadcasted_iota(jnp.int32, (1, QK_DIM), 1) // _dq
    stackmask = (qrow == qcol).astype(jnp.float32)
    vi = jax.lax.broadcasted_iota(jnp.int32, (1, DIM), 1) // _dv

    o_list = []
    for iw in range(NPATCH):
        left = OFFS[iw]
        q_p = q_ref[:, left:left + PS, :].reshape(NTOK, QK_DIM)
        k_p = k_ref[:, left:left + PS, :].reshape(NTOK, QK_DIM)
        v_p = v_ref[:, left:left + PS, :].reshape(NTOK, DIM)

        q_stack = jnp.concatenate([q_p] * HEADS, axis=0) * stackmask
        s = jax.lax.dot_general(
            q_stack, k_p, (((1,), (1,)), ((), ())),
            precision=jax.lax.Precision.DEFAULT,
            preferred_element_type=jnp.float32)
        s = s - jnp.max(s, axis=-1, keepdims=True)
        e = jnp.exp(s)
        ov = jnp.dot(e, v_p, precision=jax.lax.Precision.DEFAULT,
                     preferred_element_type=jnp.float32)
        ov = ov / jnp.sum(e, axis=-1, keepdims=True)
        o = (vi == 0).astype(jnp.float32) * ov[0:NTOK]
        for h in range(1, HEADS):
            o = o + (vi == h).astype(jnp.float32) * ov[h * NTOK:(h + 1) * NTOK]
        o_list.append(o)

    o_ref[...] = jnp.concatenate(o_list, axis=0).reshape(
        1, NPATCH * NTOK, DIM)


def _proj_kernel(o_ref, wp_ref, out_ref):
    i = pl.program_id(0)
    top = _top_of(i)

    o_row = o_ref[...].reshape(NPATCH * NTOK, DIM)
    res_row = jnp.dot(o_row, wp_ref[...],
                      preferred_element_type=jnp.float32)

    wr = _band_weights(
        top + jax.lax.broadcasted_iota(jnp.int32, (PS, 1, 1), 0))
    colw = _band_weights(
        jax.lax.broadcasted_iota(jnp.int32, (1, H, 1), 1))

    patches = []
    for iw in range(NPATCH):
        left = OFFS[iw]
        p = res_row[iw * NTOK:(iw + 1) * NTOK].reshape(PS, PS, DIM)
        patches.append(p * wr * colw[:, left:left + PS, :])

    pieces = [patches[0][:, 0:OFFS[1], :]]
    for iw in range(1, NPATCH):
        ov_w = OFFS[iw - 1] + PS - OFFS[iw]
        pieces.append(patches[iw - 1][:, PS - ov_w:PS, :]
                      + patches[iw][:, 0:ov_w, :])
        hi = (OFFS[iw + 1] - OFFS[iw]) if iw + 1 < NPATCH else PS
        pieces.append(patches[iw][:, ov_w:hi, :])
    slab = jnp.concatenate(pieces, axis=1)

    @pl.when(i == 0)
    def _first():
        out_ref[pl.ds(top, PS)] = slab

    @pl.when(jnp.logical_and(i > 0, i < NPATCH - 1))
    def _interior():
        out_ref[pl.ds(top, 2)] += slab[0:2]
        out_ref[pl.ds(top + 2, PS - 2)] = slab[2:PS]

    @pl.when(i == NPATCH - 1)
    def _last():
        out_ref[pl.ds(top, 4)] += slab[0:4]
        out_ref[pl.ds(top + 4, PS - 4)] = slab[4:PS]


def _back_kernel(y_ref, x_ref, out_ref):
    out_ref[...] = y_ref[...].T + x_ref[...]


def kernel(x, ln_g, ln_b, Wq, Wk, Wv, Wp):
    x2 = x[0].reshape(DIM, H * H)
    wq_s = Wq * (1.0 / np.sqrt(_dq))

    nrow = H * H // 14
    q, k, v = pl.pallas_call(
        _qkv_kernel,
        grid=(14,),
        in_specs=[
            pl.BlockSpec((DIM, nrow), lambda i: (0, i)),
            pl.BlockSpec((1, DIM), lambda i: (0, 0)),
            pl.BlockSpec((1, DIM), lambda i: (0, 0)),
            pl.BlockSpec((DIM, QK_DIM), lambda i: (0, 0)),
            pl.BlockSpec((DIM, QK_DIM), lambda i: (0, 0)),
            pl.BlockSpec((DIM, DIM), lambda i: (0, 0)),
        ],
        out_specs=[
            pl.BlockSpec((nrow, QK_DIM), lambda i: (i, 0)),
            pl.BlockSpec((nrow, QK_DIM), lambda i: (i, 0)),
            pl.BlockSpec((nrow, DIM), lambda i: (i, 0)),
        ],
        out_shape=[
            jax.ShapeDtypeStruct((H * H, QK_DIM), jnp.float32),
            jax.ShapeDtypeStruct((H * H, QK_DIM), jnp.float32),
            jax.ShapeDtypeStruct((H * H, DIM), jnp.float32),
        ],
    )(x2, ln_g.reshape(1, DIM), ln_b.reshape(1, DIM), wq_s, Wk, Wv)

    q4 = q.reshape(H, H, QK_DIM)
    k4 = k.reshape(H, H, QK_DIM)
    v4 = v.reshape(H, H, DIM)

    o_all = pl.pallas_call(
        _attn_kernel,
        grid=(NPATCH,),
        in_specs=[
            pl.BlockSpec((Element(PS), Element(H), Element(QK_DIM)),
                         lambda i: (_top_of(i), 0, 0)),
            pl.BlockSpec((Element(PS), Element(H), Element(QK_DIM)),
                         lambda i: (_top_of(i), 0, 0)),
            pl.BlockSpec((Element(PS), Element(H), Element(DIM)),
                         lambda i: (_top_of(i), 0, 0)),
        ],
        out_specs=pl.BlockSpec((1, NPATCH * NTOK, DIM), lambda i: (i, 0, 0)),
        out_shape=jax.ShapeDtypeStruct((NPATCH, NPATCH * NTOK, DIM),
                                       jnp.float32),
    )(q4, k4, v4)

    out_hwc = pl.pallas_call(
        _proj_kernel,
        grid=(NPATCH,),
        in_specs=[
            pl.BlockSpec((1, NPATCH * NTOK, DIM), lambda i: (i, 0, 0)),
            pl.BlockSpec((DIM, DIM), lambda i: (0, 0)),
        ],
        out_specs=pl.BlockSpec((H, H, DIM), lambda i: (0, 0, 0)),
        out_shape=jax.ShapeDtypeStruct((H, H, DIM), jnp.float32),
    )(o_all, Wp)

    out_chw = pl.pallas_call(
        _back_kernel,
        grid=(14,),
        in_specs=[
            pl.BlockSpec((nrow, DIM), lambda i: (i, 0)),
            pl.BlockSpec((DIM, nrow), lambda i: (0, i)),
        ],
        out_specs=pl.BlockSpec((DIM, nrow), lambda i: (0, i)),
        out_shape=jax.ShapeDtypeStruct((DIM, H * H), jnp.float32),
    )(out_hwc.reshape(H * H, DIM), x2)

    return out_chw.reshape(1, DIM, H, H)
